# baseline (speedup 1.0000x reference)
import functools

import jax
import jax.numpy as jnp
from jax import lax
from jax.experimental import pallas as pl
from jax.experimental.pallas import tpu as pltpu
from jax.experimental.pallas import tpu_sc as plsc

T_LEN = 8192
BATCH = 4
HID = 1024
NHEADS = 16
N = BATCH * NHEADS
R = HID // NHEADS
KNORM = 3.0
BT = 128
NSC = 32
CH = 128
NCHUNK = T_LEN * N // CH
CPW = NCHUNK // NSC
GRP = 8


def _prep_body(x_ref, tri_ref, w2_ref, b2_ref,
               sp_ref, rr_ref, rl_ref,
               wr0_ref, wr1_ref, wl0_ref, wl1_ref, carry_ref):
    i = pl.program_id(0)

    @pl.when(i == 0)
    def _():
        carry_ref[:] = jnp.zeros_like(carry_ref)

    logits = b2_ref[:]
    for b in range(BATCH):
        xb_b = x_ref[:, b, :]
        old_b = carry_ref[:, pl.ds(b * HID, HID)]
        cs_b = lax.dot_general(tri_ref[:], xb_b * (1.0 / KNORM),
                               (((1,), (0,)), ((), ())),
                               precision=lax.Precision.HIGHEST,
                               preferred_element_type=jnp.float32) + old_b
        carry_ref[:, pl.ds(b * HID, HID)] = cs_b[BT - 1:BT, :]
        shifted_b = jnp.concatenate([old_b, cs_b[:BT - 1, :]], axis=0)
        sp_ref[:, pl.ds(b * NHEADS, NHEADS), :] = jnp.concatenate(
            [shifted_b.reshape(BT, NHEADS, R),
             cs_b.reshape(BT, NHEADS, R)], axis=2)
        logits = logits + lax.dot_general(
            xb_b, w2_ref[pl.ds(b * HID, HID), :], (((1,), (0,)), ((), ())),
            preferred_element_type=jnp.float32)
    offs = jax.nn.sigmoid(logits)

    half = BT // 2
    ri2 = 2 * lax.broadcasted_iota(jnp.int32, (half, BT), 0)
    si = lax.broadcasted_iota(jnp.int32, (half, BT), 1)
    esel = (si == ri2).astype(jnp.float32)
    osel = (si == ri2 + 1).astype(jnp.float32)

    def fold(v):
        dn = (((1,), (0,)), ((), ()))
        return jnp.concatenate(
            [lax.dot_general(esel, v, dn, precision=lax.Precision.HIGHEST,
                             preferred_element_type=jnp.float32),
             lax.dot_general(osel, v, dn, precision=lax.Precision.HIGHEST,
                             preferred_element_type=jnp.float32)], axis=1)

    al = fold(offs[:, :N])
    ar = fold(offs[:, N:])

    rowi = lax.broadcasted_iota(jnp.int32, (half, 2 * N), 0)
    coli = lax.broadcasted_iota(jnp.int32, (half, 2 * N), 1)
    parity = (coli >= N).astype(jnp.int32)
    tf = (2 * rowi + parity).astype(jnp.float32) + jnp.float32(i) * BT
    ncol = coli - N * parity
    tm1 = float(T_LEN - 1)

    left_len = 1.0 + al * jnp.maximum(tf - 1.0, 0.0)
    lpos = tf - left_len - 1.0
    right_len = 1.0 + ar * jnp.maximum((tm1 - tf) - 1.0, 0.0)
    rpos = jnp.minimum(tf + right_len, tm1)

    def mk(pos):
        p = jnp.clip(pos, -1.0, tm1)
        fl = jnp.floor(p)
        fr = p - fl
        i0 = fl.astype(jnp.int32)
        last = i0 == (T_LEN - 1)
        row = jnp.minimum(i0 + 1, T_LEN - 1) * N + ncol
        w0 = jnp.where(last | (i0 < 0), 0.0, 1.0 - fr)
        w1 = jnp.where(last, 1.0, fr)
        return row, w0, w1

    rowr, w0r, w1r = mk(rpos)
    rowl, w0l, w1l = mk(lpos)
    rr_ref[:] = rowr
    rl_ref[:] = rowl
    wr0_ref[:] = w0r
    wr1_ref[:] = w1r
    wl0_ref[:] = -w0l
    wl1_ref[:] = -w1l


def _tc_prep(x2, tri, w2, b2):
    f32 = jnp.float32
    i32 = jnp.int32
    outs = ([jax.ShapeDtypeStruct((T_LEN, N, 2 * R), f32)]
            + [jax.ShapeDtypeStruct((NCHUNK, CH), i32)] * 2
            + [jax.ShapeDtypeStruct((NCHUNK, CH), f32)] * 4)
    return pl.pallas_call(
        _prep_body,
        grid=(T_LEN // BT,),
        in_specs=[
            pl.BlockSpec((BT, BATCH, HID), lambda i: (i, 0, 0)),
            pl.BlockSpec((BT, BT), lambda i: (0, 0)),
            pl.BlockSpec((N * R, 2 * N), lambda i: (0, 0)),
            pl.BlockSpec((1, 2 * N), lambda i: (0, 0)),
        ],
        out_specs=([pl.BlockSpec((BT, N, 2 * R), lambda i: (i, 0, 0))]
                   + [pl.BlockSpec((BT // 2, CH), lambda i: (i, 0))] * 6),
        out_shape=outs,
        scratch_shapes=[pltpu.VMEM((1, N * R), f32)],
        compiler_params=pltpu.CompilerParams(
            dimension_semantics=("arbitrary",)),
    )(x2, tri, w2, b2)


def _sc_body(sp, rr, rl, w0r, w1r, w0l, w1l, out,
             idxb, wtsb, gb, outv,
             sstage, sg0, sg1, so0, so1):
    rrefs = (rr, rl)
    wrefs = (w0r, w1r, w0l, w1l)
    sgs = (sg0, sg1)
    sos = (so0, so1)
    wid = lax.axis_index("s") * 2 + lax.axis_index("c")
    cbase = wid * CPW

    def combine(par, c):
        def pgrp(q, carry3):
            wv = [wtsb[j, c, pl.ds(q * 16, 16)] for j in range(4)]
            for k in range(16):
                p = q * 16 + k
                ws = [wv[j][k] for j in range(4)]
                for g in range(R // 16):
                    acc = gb[par, 0, p, pl.ds(g * 16, 16)] * ws[0]
                    acc = acc + gb[par, 0, p, pl.ds(R + g * 16, 16)] * ws[1]
                    acc = acc + gb[par, 1, p, pl.ds(g * 16, 16)] * ws[2]
                    acc = acc + gb[par, 1, p, pl.ds(R + g * 16, 16)] * ws[3]
                    outv[par, p, pl.ds(g * 16, 16)] = acc
            return carry3

        lax.fori_loop(0, CH // 16, pgrp, 0)

    def group(o, carry):
        crow0 = cbase + o * GRP
        st = [pltpu.async_copy(rrefs[j].at[pl.ds(crow0, GRP)],
                               idxb.at[j], sstage) for j in range(2)]
        st += [pltpu.async_copy(wrefs[j].at[pl.ds(crow0, GRP)],
                                wtsb.at[j], sstage) for j in range(4)]
        for s in st:
            s.wait()

        g_prev = [pltpu.async_copy(sp.at[idxb.at[j, 0]], gb.at[0, j], sgs[0])
                  for j in range(2)]
        stores = [None, None]
        for c in range(GRP):
            par = c % 2
            npar = 1 - par
            if c + 1 < GRP:
                g_next = [pltpu.async_copy(sp.at[idxb.at[j, c + 1]],
                                           gb.at[npar, j], sgs[npar])
                          for j in range(2)]
            for gcp in g_prev:
                gcp.wait()
            if stores[par] is not None:
                stores[par].wait()
            combine(par, c)
            stores[par] = pltpu.async_copy(
                outv.at[par], out.at[pl.ds((crow0 + c) * CH, CH)], sos[par])
            if c + 1 < GRP:
                g_prev = g_next
        for s in stores:
            s.wait()
        return carry

    lax.fori_loop(0, CPW // GRP, group, 0)


def _sc_combine(sp, rows, wts):
    mesh = plsc.VectorSubcoreMesh(core_axis_name="c", subcore_axis_name="s")
    kern = pl.kernel(
        _sc_body,
        out_type=jax.ShapeDtypeStruct((T_LEN * N, R), jnp.float32),
        mesh=mesh,
        scratch_types=[
            pltpu.VMEM((2, GRP, CH), jnp.int32),
            pltpu.VMEM((4, GRP, CH), jnp.float32),
            pltpu.VMEM((2, 2, CH, 2 * R), jnp.float32),
            pltpu.VMEM((2, CH, R), jnp.float32),
            pltpu.SemaphoreType.DMA,
            pltpu.SemaphoreType.DMA,
            pltpu.SemaphoreType.DMA,
            pltpu.SemaphoreType.DMA,
            pltpu.SemaphoreType.DMA,
        ])
    return kern(sp, *rows, *wts)


def kernel(x, W_off, b_off):
    eye = jnp.eye(BATCH, dtype=jnp.float32)
    w2 = jnp.concatenate([jnp.kron(eye, W_off[:NHEADS].T),
                          jnp.kron(eye, W_off[NHEADS:].T)], axis=1)
    b2 = jnp.concatenate([jnp.tile(b_off[:NHEADS], BATCH),
                          jnp.tile(b_off[NHEADS:], BATCH)])[None, :]
    tri = jnp.tril(jnp.ones((BT, BT), jnp.float32))
    sp, rr, rl, wr0, wr1, wl0, wl1 = _tc_prep(x, tri, w2, b2)
    sp2 = sp.reshape(T_LEN * N, 2 * R)
    out = _sc_combine(sp2, [rr, rl], [wr0, wr1, wl0, wl1])
    return out.reshape(T_LEN, BATCH, HID)


# speedup vs baseline: 1.0031x; 1.0031x over previous
import functools

import jax
import jax.numpy as jnp
from jax import lax
from jax.experimental import pallas as pl
from jax.experimental.pallas import tpu as pltpu
from jax.experimental.pallas import tpu_sc as plsc

T_LEN = 8192
BATCH = 4
HID = 1024
NHEADS = 16
N = BATCH * NHEADS
R = HID // NHEADS
KNORM = 3.0
BT = 256
NSC = 32
CH = 128
NCHUNK = T_LEN * N // CH
CPW = NCHUNK // NSC
GRP = 8


def _prep_body(x_ref, tri_ref, w2_ref, b2_ref,
               sp_ref, rr_ref, rl_ref,
               wr0_ref, wr1_ref, wl0_ref, wl1_ref, carry_ref):
    i = pl.program_id(0)

    @pl.when(i == 0)
    def _():
        carry_ref[:] = jnp.zeros_like(carry_ref)

    logits = b2_ref[:]
    for b in range(BATCH):
        xb_b = x_ref[:, b, :]
        old_b = carry_ref[:, pl.ds(b * HID, HID)]
        cs_b = lax.dot_general(tri_ref[:], xb_b * (1.0 / KNORM),
                               (((1,), (0,)), ((), ())),
                               precision=lax.Precision.HIGHEST,
                               preferred_element_type=jnp.float32) + old_b
        carry_ref[:, pl.ds(b * HID, HID)] = cs_b[BT - 1:BT, :]
        shifted_b = jnp.concatenate([old_b, cs_b[:BT - 1, :]], axis=0)
        sp_ref[:, pl.ds(b * NHEADS, NHEADS), :] = jnp.concatenate(
            [shifted_b.reshape(BT, NHEADS, R),
             cs_b.reshape(BT, NHEADS, R)], axis=2)
        logits = logits + lax.dot_general(
            xb_b, w2_ref[pl.ds(b * HID, HID), :], (((1,), (0,)), ((), ())),
            preferred_element_type=jnp.float32)
    offs = jax.nn.sigmoid(logits)

    half = BT // 2
    ri2 = 2 * lax.broadcasted_iota(jnp.int32, (half, BT), 0)
    si = lax.broadcasted_iota(jnp.int32, (half, BT), 1)
    esel = (si == ri2).astype(jnp.float32)
    osel = (si == ri2 + 1).astype(jnp.float32)

    def fold(v):
        dn = (((1,), (0,)), ((), ()))
        return jnp.concatenate(
            [lax.dot_general(esel, v, dn, precision=lax.Precision.HIGHEST,
                             preferred_element_type=jnp.float32),
             lax.dot_general(osel, v, dn, precision=lax.Precision.HIGHEST,
                             preferred_element_type=jnp.float32)], axis=1)

    al = fold(offs[:, :N])
    ar = fold(offs[:, N:])

    rowi = lax.broadcasted_iota(jnp.int32, (half, 2 * N), 0)
    coli = lax.broadcasted_iota(jnp.int32, (half, 2 * N), 1)
    parity = (coli >= N).astype(jnp.int32)
    tf = (2 * rowi + parity).astype(jnp.float32) + jnp.float32(i) * BT
    ncol = coli - N * parity
    tm1 = float(T_LEN - 1)

    left_len = 1.0 + al * jnp.maximum(tf - 1.0, 0.0)
    lpos = tf - left_len - 1.0
    right_len = 1.0 + ar * jnp.maximum((tm1 - tf) - 1.0, 0.0)
    rpos = jnp.minimum(tf + right_len, tm1)

    def mk(pos):
        p = jnp.clip(pos, -1.0, tm1)
        fl = jnp.floor(p)
        fr = p - fl
        i0 = fl.astype(jnp.int32)
        last = i0 == (T_LEN - 1)
        row = jnp.minimum(i0 + 1, T_LEN - 1) * N + ncol
        w0 = jnp.where(last | (i0 < 0), 0.0, 1.0 - fr)
        w1 = jnp.where(last, 1.0, fr)
        return row, w0, w1

    rowr, w0r, w1r = mk(rpos)
    rowl, w0l, w1l = mk(lpos)
    rr_ref[:] = rowr
    rl_ref[:] = rowl
    wr0_ref[:] = w0r
    wr1_ref[:] = w1r
    wl0_ref[:] = -w0l
    wl1_ref[:] = -w1l


def _tc_prep(x2, tri, w2, b2):
    f32 = jnp.float32
    i32 = jnp.int32
    outs = ([jax.ShapeDtypeStruct((T_LEN, N, 2 * R), f32)]
            + [jax.ShapeDtypeStruct((NCHUNK, CH), i32)] * 2
            + [jax.ShapeDtypeStruct((NCHUNK, CH), f32)] * 4)
    return pl.pallas_call(
        _prep_body,
        grid=(T_LEN // BT,),
        in_specs=[
            pl.BlockSpec((BT, BATCH, HID), lambda i: (i, 0, 0)),
            pl.BlockSpec((BT, BT), lambda i: (0, 0)),
            pl.BlockSpec((N * R, 2 * N), lambda i: (0, 0)),
            pl.BlockSpec((1, 2 * N), lambda i: (0, 0)),
        ],
        out_specs=([pl.BlockSpec((BT, N, 2 * R), lambda i: (i, 0, 0))]
                   + [pl.BlockSpec((BT // 2, CH), lambda i: (i, 0))] * 6),
        out_shape=outs,
        scratch_shapes=[pltpu.VMEM((1, N * R), f32)],
        compiler_params=pltpu.CompilerParams(
            dimension_semantics=("arbitrary",)),
    )(x2, tri, w2, b2)


def _sc_body(sp, rr, rl, w0r, w1r, w0l, w1l, out,
             idxb, wtsb, gb, outv,
             sstage, sg0, sg1, so0, so1):
    rrefs = (rr, rl)
    wrefs = (w0r, w1r, w0l, w1l)
    sgs = (sg0, sg1)
    sos = (so0, so1)
    wid = lax.axis_index("s") * 2 + lax.axis_index("c")
    cbase = wid * CPW

    def combine(par, c):
        def pgrp(q, carry3):
            wv = [wtsb[j, c, pl.ds(q * 16, 16)] for j in range(4)]
            for k in range(16):
                p = q * 16 + k
                ws = [wv[j][k] for j in range(4)]
                for g in range(R // 16):
                    acc = gb[par, 0, p, pl.ds(g * 16, 16)] * ws[0]
                    acc = acc + gb[par, 0, p, pl.ds(R + g * 16, 16)] * ws[1]
                    acc = acc + gb[par, 1, p, pl.ds(g * 16, 16)] * ws[2]
                    acc = acc + gb[par, 1, p, pl.ds(R + g * 16, 16)] * ws[3]
                    outv[par, p, pl.ds(g * 16, 16)] = acc
            return carry3

        lax.fori_loop(0, CH // 16, pgrp, 0)

    def group(o, carry):
        crow0 = cbase + o * GRP
        st = [pltpu.async_copy(rrefs[j].at[pl.ds(crow0, GRP)],
                               idxb.at[j], sstage) for j in range(2)]
        st += [pltpu.async_copy(wrefs[j].at[pl.ds(crow0, GRP)],
                                wtsb.at[j], sstage) for j in range(4)]
        for s in st:
            s.wait()

        g_prev = [pltpu.async_copy(sp.at[idxb.at[j, 0]], gb.at[0, j], sgs[0])
                  for j in range(2)]
        stores = [None, None]
        for c in range(GRP):
            par = c % 2
            npar = 1 - par
            if c + 1 < GRP:
                g_next = [pltpu.async_copy(sp.at[idxb.at[j, c + 1]],
                                           gb.at[npar, j], sgs[npar])
                          for j in range(2)]
            for gcp in g_prev:
                gcp.wait()
            if stores[par] is not None:
                stores[par].wait()
            combine(par, c)
            stores[par] = pltpu.async_copy(
                outv.at[par], out.at[pl.ds((crow0 + c) * CH, CH)], sos[par])
            if c + 1 < GRP:
                g_prev = g_next
        for s in stores:
            s.wait()
        return carry

    lax.fori_loop(0, CPW // GRP, group, 0)


def _sc_combine(sp, rows, wts):
    mesh = plsc.VectorSubcoreMesh(core_axis_name="c", subcore_axis_name="s")
    kern = pl.kernel(
        _sc_body,
        out_type=jax.ShapeDtypeStruct((T_LEN * N, R), jnp.float32),
        mesh=mesh,
        scratch_types=[
            pltpu.VMEM((2, GRP, CH), jnp.int32),
            pltpu.VMEM((4, GRP, CH), jnp.float32),
            pltpu.VMEM((2, 2, CH, 2 * R), jnp.float32),
            pltpu.VMEM((2, CH, R), jnp.float32),
            pltpu.SemaphoreType.DMA,
            pltpu.SemaphoreType.DMA,
            pltpu.SemaphoreType.DMA,
            pltpu.SemaphoreType.DMA,
            pltpu.SemaphoreType.DMA,
        ])
    return kern(sp, *rows, *wts)


def kernel(x, W_off, b_off):
    eye = jnp.eye(BATCH, dtype=jnp.float32)
    w2 = jnp.concatenate([jnp.kron(eye, W_off[:NHEADS].T),
                          jnp.kron(eye, W_off[NHEADS:].T)], axis=1)
    b2 = jnp.concatenate([jnp.tile(b_off[:NHEADS], BATCH),
                          jnp.tile(b_off[NHEADS:], BATCH)])[None, :]
    tri = jnp.tril(jnp.ones((BT, BT), jnp.float32))
    sp, rr, rl, wr0, wr1, wl0, wl1 = _tc_prep(x, tri, w2, b2)
    sp2 = sp.reshape(T_LEN * N, 2 * R)
    out = _sc_combine(sp2, [rr, rl], [wr0, wr1, wl0, wl1])
    return out.reshape(T_LEN, BATCH, HID)
